# baseline (speedup 1.0000x reference)
import jax
import jax.numpy as jnp
import numpy as np
from jax import lax
from jax.experimental import pallas as pl
from jax.experimental.pallas import tpu as pltpu
from jax.experimental.pallas import tpu_sc as plsc

N = 10000
E = 320000
IN_DIM = 128
H = 8
D = 16
HD = H * D
LEAKY_SLOPE = 0.2

NC = 2
NS = 16
NW = NC * NS
EPW = E // NW
K = 40
NCHUNK = EPW // K
N_PAD = 10240
RPT = N_PAD // NS
ZB = 128

_f32 = jnp.float32
_i32 = jnp.int32


TBL = HD + 32


def _mm_body(x_ref, wall_ref, whb_ref, sdst_ref):
    xb = x_ref[...]
    y = jnp.dot(xb, wall_ref[...], preferred_element_type=_f32)
    whb_ref[...] = y[:, :TBL].astype(jnp.bfloat16)
    sdst_ref[...] = y[:, TBL:]


def _precompute(x, wall, blk=400):
    grid = (N // blk,)
    return pl.pallas_call(
        _mm_body,
        grid=grid,
        in_specs=[
            pl.BlockSpec((blk, IN_DIM), lambda i: (i, 0)),
            pl.BlockSpec((IN_DIM, TBL + 16), lambda i: (0, 0)),
        ],
        out_specs=[
            pl.BlockSpec((blk, TBL), lambda i: (i, 0)),
            pl.BlockSpec((blk, 16), lambda i: (i, 0)),
        ],
        out_shape=[
            jax.ShapeDtypeStruct((N, TBL), jnp.bfloat16),
            jax.ShapeDtypeStruct((N, 16), _f32),
        ],
    )(x, wall)


HDX = HD + 16


PH = NCHUNK // 2


def _edge_body(src_hbm, dst_hbm, sdst_hbm, whb_hbm,
               u_hbm,
               sidx_all, didx_all, dbuf0, dbuf1,
               whbuf0, whbuf1, obuf0, obuf1,
               u_sh,
               semw0, semw1, semd0, semd1, semo0, semo1):
    dbuf = (dbuf0, dbuf1)
    whbuf = (whbuf0, whbuf1)
    obuf = (obuf0, obuf1)
    semw = (semw0, semw1)
    semd = (semd0, semd1)
    semo = (semo0, semo1)
    c = lax.axis_index("c")
    s = lax.axis_index("s")
    w = c * NS + s
    rows = pl.ds(s * RPT, RPT)

    def zrow(i, carry):
        for j in range(HDX // 16):
            obuf0[i, pl.ds(16 * j, 16)] = jnp.zeros((16,), _f32)
        return carry
    lax.fori_loop(0, K, zrow, 0)
    for j in range(RPT // K):
        pltpu.sync_copy(obuf0, u_sh.at[pl.ds(s * RPT + j * K, K)])
    plsc.subcore_barrier()

    def issue(ci, b):
        sidx = sidx_all.at[ci]
        didx = didx_all.at[ci]
        pltpu.async_copy(whb_hbm.at[sidx], whbuf[b], semw[b])
        pltpu.async_copy(sdst_hbm.at[didx], dbuf[b], semd[b])

    def wait(ci, b):
        sidx = sidx_all.at[ci]
        didx = didx_all.at[ci]
        pltpu.make_async_copy(whb_hbm.at[sidx], whbuf[b], semw[b]).wait()
        pltpu.make_async_copy(sdst_hbm.at[didx], dbuf[b], semd[b]).wait()

    def wait_scatter(ci, b):
        pltpu.make_async_copy(
            obuf[b], u_sh.at[didx_all.at[ci]], semo[b]).wait()

    def process(ci, b):
        db, wb, ob = dbuf[b], whbuf[b], obuf[b]

        @pl.when(ci >= 2)
        def _drain_prev():
            wait_scatter(ci, b)

        @plsc.parallel_loop(0, K, 1, unroll=8)
        def _edge(k):
            sv = wb[k, pl.ds(HD, 32)]
            slo, _shi = plsc.unpack(sv, format=plsc.PackFormat.INTERLEAVED)
            e = slo + db[k, :]
            e = jnp.where(e >= 0.0, e, LEAKY_SLOPE * e)
            fexp = jnp.exp(e)
            ob[k, pl.ds(HD, 16)] = fexp
            for t in range(HD // 32):
                v = wb[k, pl.ds(32 * t, 32)]
                lo, hi = plsc.unpack(v, format=plsc.PackFormat.INTERLEAVED)
                ob[k, pl.ds(32 * t, 16)] = lo * fexp
                ob[k, pl.ds(32 * t + 16, 16)] = hi * fexp

        pltpu.async_copy(ob, u_sh.at[didx_all.at[ci]], semo[b], add=True)

    for p in range(2):
        pltpu.sync_copy(src_hbm.at[w, pl.ds(p * PH, PH)], sidx_all)
        pltpu.sync_copy(dst_hbm.at[w, pl.ds(p * PH, PH)], didx_all)
        issue(0, 0)

        def outer(i, carry):
            for b in range(2):
                ci = 2 * i + b

                @pl.when(ci + 1 < PH)
                def _issue_next():
                    issue(ci + 1, 1 - b)
                wait(ci, b)
                process(ci, b)
            return carry
        lax.fori_loop(0, PH // 2, outer, 0)
        wait(PH - 1, 0)
        process(PH - 1, 0)
        wait_scatter(PH - 1, 0)
        wait_scatter(PH - 2, 1)

    plsc.subcore_barrier()
    pltpu.sync_copy(u_sh.at[rows], u_hbm.at[c, rows])


def _edge_phase(src, dst, sdst, whb):
    f = pl.kernel(
        _edge_body,
        out_type=jax.ShapeDtypeStruct((NC, N_PAD, HDX), _f32),
        mesh=plsc.VectorSubcoreMesh(core_axis_name="c", subcore_axis_name="s"),
        compiler_params=pltpu.CompilerParams(
            use_tc_tiling_on_sc=False, needs_layout_passes=False),
        scratch_types=[
            pltpu.VMEM((PH, K), _i32),
            pltpu.VMEM((PH, K), _i32),
            pltpu.VMEM((K, 16), _f32),
            pltpu.VMEM((K, 16), _f32),
            pltpu.VMEM((K, TBL), jnp.bfloat16),
            pltpu.VMEM((K, TBL), jnp.bfloat16),
            pltpu.VMEM((K, HDX), _f32),
            pltpu.VMEM((K, HDX), _f32),
            pltpu.VMEM_SHARED((N_PAD, HDX), _f32),
            pltpu.SemaphoreType.DMA,
            pltpu.SemaphoreType.DMA,
            pltpu.SemaphoreType.DMA,
            pltpu.SemaphoreType.DMA,
            pltpu.SemaphoreType.DMA,
            pltpu.SemaphoreType.DMA,
        ],
    )
    return f(src, dst, sdst, whb)


def _final_body(u0_ref, u1_ref, p_ref, b_ref, out_ref):
    t = u0_ref[0] + u1_ref[0]
    r = 1.0 / (t[:, HD:] + 1e-16)
    rr = jnp.concatenate([r] * (HD // 16), axis=1)
    y = t[:, :HD] * rr
    out_ref[...] = (
        jnp.dot(y, p_ref[...], preferred_element_type=_f32) + b_ref[...]
    )


def _finalize(u, perm_mat, bias, blk=400):
    grid = (N // blk,)
    return pl.pallas_call(
        _final_body,
        grid=grid,
        in_specs=[
            pl.BlockSpec((1, blk, HDX), lambda i: (0, i, 0)),
            pl.BlockSpec((1, blk, HDX), lambda i: (1, i, 0)),
            pl.BlockSpec((HD, HD), lambda i: (0, 0)),
            pl.BlockSpec((1, HD), lambda i: (0, 0)),
        ],
        out_specs=pl.BlockSpec((blk, HD), lambda i: (i, 0)),
        out_shape=jax.ShapeDtypeStruct((N, HD), _f32),
    )(u, u, perm_mat, bias)


_j = np.arange(HD)
_DM_FROM_STD = (_j % H) * D + _j // H
_STD_FROM_DM = (_j % D) * H + _j // D
_PERM = np.zeros((HD, HD), dtype=np.float32)
_PERM[_STD_FROM_DM, _j] = 1.0
_ILV = 32 * (_j // 32) + (_j % 32) // 2 + 16 * (_j % 2)
_PERM_IN = np.zeros((HD, HD), dtype=np.float32)
_PERM_IN[_DM_FROM_STD[_ILV], _j] = 1.0


@jax.jit
def kernel(x, src, dst, W, a, bias):
    wp = W @ jnp.asarray(_PERM_IN)
    a1 = a[:, :D]
    a2 = a[:, D:]
    eye = np.equal.outer(np.arange(H), np.arange(H)).astype(np.float32)
    msrc = (a1[:, :, None] * eye[:, None, :]).reshape(HD, H)
    mdst = (a2[:, :, None] * eye[:, None, :]).reshape(HD, H)
    wm = jnp.concatenate(
        [W @ msrc, W @ msrc, W @ mdst, W @ mdst], axis=1)
    q = np.zeros((32, 32), dtype=np.float32)
    cols = np.arange(32)
    q[cols[::2] // 2 * 0 + (cols[::2] // 2), cols[::2]] = 1.0
    wall = jnp.concatenate(
        [wp, wm @ jnp.asarray(q), wm[:, 16:]], axis=1)

    whb, sdst = _precompute(x, wall)
    u = _edge_phase(
        src.reshape(NW, NCHUNK, K), dst.reshape(NW, NCHUNK, K),
        sdst, whb)
    return _finalize(u, jnp.asarray(_PERM), bias.reshape(1, HD))


# speedup vs baseline: 1.0301x; 1.0301x over previous
import jax
import jax.numpy as jnp
import numpy as np
from jax import lax
from jax.experimental import pallas as pl
from jax.experimental.pallas import tpu as pltpu
from jax.experimental.pallas import tpu_sc as plsc

N = 10000
E = 320000
IN_DIM = 128
H = 8
D = 16
HD = H * D
LEAKY_SLOPE = 0.2

NC = 2
NS = 16
NW = NC * NS
EPW = E // NW
K = 40
NCHUNK = EPW // K
N_PAD = 10240
RPT = N_PAD // NS
ZB = 128

_f32 = jnp.float32
_i32 = jnp.int32


def _mm_body(x_ref, wp_ref, wm_ref, whb_ref, ssrc_ref, sdst_ref):
    xb = x_ref[...]
    wh = jnp.dot(xb, wp_ref[...], preferred_element_type=_f32)
    s = jnp.dot(xb, wm_ref[...], preferred_element_type=_f32)
    whb_ref[...] = wh.astype(jnp.bfloat16)
    ssrc_ref[...] = s[:, :16]
    sdst_ref[...] = s[:, 16:]


def _precompute(x, wp, wm, blk=400):
    grid = (N // blk,)
    return pl.pallas_call(
        _mm_body,
        grid=grid,
        in_specs=[
            pl.BlockSpec((blk, IN_DIM), lambda i: (i, 0)),
            pl.BlockSpec((IN_DIM, HD), lambda i: (0, 0)),
            pl.BlockSpec((IN_DIM, 32), lambda i: (0, 0)),
        ],
        out_specs=[
            pl.BlockSpec((blk, HD), lambda i: (i, 0)),
            pl.BlockSpec((blk, 16), lambda i: (i, 0)),
            pl.BlockSpec((blk, 16), lambda i: (i, 0)),
        ],
        out_shape=[
            jax.ShapeDtypeStruct((N, HD), jnp.bfloat16),
            jax.ShapeDtypeStruct((N, 16), _f32),
            jax.ShapeDtypeStruct((N, 16), _f32),
        ],
    )(x, wp, wm)


HDX = HD + 16


PH = NCHUNK // 2


def _edge_body(src_hbm, dst_hbm, ssrc_hbm, sdst_hbm, whb_hbm,
               u_hbm,
               sidx_all, didx_all, sbuf0, sbuf1, dbuf0, dbuf1,
               whbuf0, whbuf1, obuf0, obuf1,
               u_sh,
               semw0, semw1, sems0, sems1, semd0, semd1, semo0, semo1):
    sbuf = (sbuf0, sbuf1)
    dbuf = (dbuf0, dbuf1)
    whbuf = (whbuf0, whbuf1)
    obuf = (obuf0, obuf1)
    semw = (semw0, semw1)
    sems = (sems0, sems1)
    semd = (semd0, semd1)
    semo = (semo0, semo1)
    c = lax.axis_index("c")
    s = lax.axis_index("s")
    w = c * NS + s
    rows = pl.ds(s * RPT, RPT)

    def zrow(i, carry):
        for j in range(HDX // 16):
            obuf0[i, pl.ds(16 * j, 16)] = jnp.zeros((16,), _f32)
        return carry
    lax.fori_loop(0, K, zrow, 0)
    for j in range(RPT // K):
        pltpu.sync_copy(obuf0, u_sh.at[pl.ds(s * RPT + j * K, K)])
    plsc.subcore_barrier()

    def issue(ci, b):
        sidx = sidx_all.at[ci]
        didx = didx_all.at[ci]
        pltpu.async_copy(whb_hbm.at[sidx], whbuf[b], semw[b])
        pltpu.async_copy(ssrc_hbm.at[sidx], sbuf[b], sems[b])
        pltpu.async_copy(sdst_hbm.at[didx], dbuf[b], semd[b])

    def wait(ci, b):
        sidx = sidx_all.at[ci]
        didx = didx_all.at[ci]
        pltpu.make_async_copy(whb_hbm.at[sidx], whbuf[b], semw[b]).wait()
        pltpu.make_async_copy(ssrc_hbm.at[sidx], sbuf[b], sems[b]).wait()
        pltpu.make_async_copy(sdst_hbm.at[didx], dbuf[b], semd[b]).wait()

    def wait_scatter(ci, b):
        pltpu.make_async_copy(
            obuf[b], u_sh.at[didx_all.at[ci]], semo[b]).wait()

    def process(ci, b):
        sb, db, wb, ob = sbuf[b], dbuf[b], whbuf[b], obuf[b]

        @pl.when(ci >= 2)
        def _drain_prev():
            wait_scatter(ci, b)

        @plsc.parallel_loop(0, K, 1, unroll=8)
        def _edge(k):
            e = sb[k, :] + db[k, :]
            e = jnp.where(e >= 0.0, e, LEAKY_SLOPE * e)
            fexp = jnp.exp(e)
            ob[k, pl.ds(HD, 16)] = fexp
            for t in range(HD // 32):
                v = wb[k, pl.ds(32 * t, 32)]
                lo, hi = plsc.unpack(v, format=plsc.PackFormat.INTERLEAVED)
                ob[k, pl.ds(32 * t, 16)] = lo * fexp
                ob[k, pl.ds(32 * t + 16, 16)] = hi * fexp

        pltpu.async_copy(ob, u_sh.at[didx_all.at[ci]], semo[b], add=True)

    for p in range(2):
        pltpu.sync_copy(src_hbm.at[w, pl.ds(p * PH, PH)], sidx_all)
        pltpu.sync_copy(dst_hbm.at[w, pl.ds(p * PH, PH)], didx_all)
        issue(0, 0)

        def outer(i, carry):
            for b in range(2):
                ci = 2 * i + b

                @pl.when(ci + 1 < PH)
                def _issue_next():
                    issue(ci + 1, 1 - b)
                wait(ci, b)
                process(ci, b)
            return carry
        lax.fori_loop(0, PH // 2, outer, 0)
        wait(PH - 1, 0)
        process(PH - 1, 0)
        wait_scatter(PH - 1, 0)
        wait_scatter(PH - 2, 1)

    plsc.subcore_barrier()
    pltpu.sync_copy(u_sh.at[rows], u_hbm.at[c, rows])


def _edge_phase(src, dst, ssrc, sdst, whb):
    f = pl.kernel(
        _edge_body,
        out_type=jax.ShapeDtypeStruct((NC, N_PAD, HDX), _f32),
        mesh=plsc.VectorSubcoreMesh(core_axis_name="c", subcore_axis_name="s"),
        compiler_params=pltpu.CompilerParams(
            use_tc_tiling_on_sc=False, needs_layout_passes=False),
        scratch_types=[
            pltpu.VMEM((PH, K), _i32),
            pltpu.VMEM((PH, K), _i32),
            pltpu.VMEM((K, 16), _f32),
            pltpu.VMEM((K, 16), _f32),
            pltpu.VMEM((K, 16), _f32),
            pltpu.VMEM((K, 16), _f32),
            pltpu.VMEM((K, HD), jnp.bfloat16),
            pltpu.VMEM((K, HD), jnp.bfloat16),
            pltpu.VMEM((K, HDX), _f32),
            pltpu.VMEM((K, HDX), _f32),
            pltpu.VMEM_SHARED((N_PAD, HDX), _f32),
            pltpu.SemaphoreType.DMA,
            pltpu.SemaphoreType.DMA,
            pltpu.SemaphoreType.DMA,
            pltpu.SemaphoreType.DMA,
            pltpu.SemaphoreType.DMA,
            pltpu.SemaphoreType.DMA,
            pltpu.SemaphoreType.DMA,
            pltpu.SemaphoreType.DMA,
        ],
    )
    return f(src, dst, ssrc, sdst, whb)


def _final_body(u0_ref, u1_ref, p_ref, b_ref, out_ref):
    t = u0_ref[0] + u1_ref[0]
    r = 1.0 / (t[:, HD:] + 1e-16)
    rr = jnp.concatenate([r] * (HD // 16), axis=1)
    y = t[:, :HD] * rr
    out_ref[...] = (
        jnp.dot(y, p_ref[...], preferred_element_type=_f32) + b_ref[...]
    )


def _finalize(u, perm_mat, bias, blk=400):
    grid = (N // blk,)
    return pl.pallas_call(
        _final_body,
        grid=grid,
        in_specs=[
            pl.BlockSpec((1, blk, HDX), lambda i: (0, i, 0)),
            pl.BlockSpec((1, blk, HDX), lambda i: (1, i, 0)),
            pl.BlockSpec((HD, HD), lambda i: (0, 0)),
            pl.BlockSpec((1, HD), lambda i: (0, 0)),
        ],
        out_specs=pl.BlockSpec((blk, HD), lambda i: (i, 0)),
        out_shape=jax.ShapeDtypeStruct((N, HD), _f32),
    )(u, u, perm_mat, bias)


_j = np.arange(HD)
_DM_FROM_STD = (_j % H) * D + _j // H
_STD_FROM_DM = (_j % D) * H + _j // D
_PERM = np.zeros((HD, HD), dtype=np.float32)
_PERM[_STD_FROM_DM, _j] = 1.0
_ILV = 32 * (_j // 32) + (_j % 32) // 2 + 16 * (_j % 2)
_PERM_IN = np.zeros((HD, HD), dtype=np.float32)
_PERM_IN[_DM_FROM_STD[_ILV], _j] = 1.0


@jax.jit
def kernel(x, src, dst, W, a, bias):
    wp = W @ jnp.asarray(_PERM_IN)
    a1 = a[:, :D]
    a2 = a[:, D:]
    eye = np.equal.outer(np.arange(H), np.arange(H)).astype(np.float32)
    msrc = (a1[:, :, None] * eye[:, None, :]).reshape(HD, H)
    mdst = (a2[:, :, None] * eye[:, None, :]).reshape(HD, H)
    wm = jnp.concatenate(
        [W @ msrc, W @ msrc, W @ mdst, W @ mdst], axis=1)

    whb, ssrc, sdst = _precompute(x, wp, wm)
    u = _edge_phase(
        src.reshape(NW, NCHUNK, K), dst.reshape(NW, NCHUNK, K),
        ssrc, sdst, whb)
    return _finalize(u, jnp.asarray(_PERM), bias.reshape(1, HD))
